# baseline (speedup 1.0000x reference)
import jax
import jax.numpy as jnp
from jax.experimental import pallas as pl
from jax.experimental.pallas import tpu as pltpu

_ROT1 = (13, 15, 26, 6)
_ROT2 = (17, 29, 16, 24)


def _four_rounds(x0, x1, rots):
    for r in rots:
        x0 = x0 + x1
        x1 = (x1 << jnp.uint32(r)) | (x1 >> jnp.uint32(32 - r))
        x1 = x1 ^ x0
    return x0, x1


def _threefry_bits(k0, k1, ctr):
    ks2 = k0 ^ k1 ^ jnp.uint32(0x1BD11BDA)
    x0 = k0
    x1 = ctr + k1
    x0, x1 = _four_rounds(x0, x1, _ROT1)
    x0 = x0 + k1
    x1 = x1 + (ks2 + jnp.uint32(1))
    x0, x1 = _four_rounds(x0, x1, _ROT2)
    x0 = x0 + ks2
    x1 = x1 + (k0 + jnp.uint32(2))
    x0, x1 = _four_rounds(x0, x1, _ROT1)
    x0 = x0 + k0
    x1 = x1 + (k1 + jnp.uint32(3))
    x0, x1 = _four_rounds(x0, x1, _ROT2)
    x0 = x0 + k1
    x1 = x1 + (ks2 + jnp.uint32(4))
    x0, x1 = _four_rounds(x0, x1, _ROT1)
    x0 = x0 + ks2
    x1 = x1 + (k0 + jnp.uint32(5))
    return x0 ^ x1


def _bits_to_unif(bits):
    f = jax.lax.bitcast_convert_type(
        (bits >> jnp.uint32(9)) | jnp.uint32(0x3F800000), jnp.float32
    )
    return f - 1.0


def _count_body(x_ref, o_ref):
    @pl.when(pl.program_id(0) == 0)
    def _():
        o_ref[0, 0] = jnp.int32(0)

    o_ref[0, 0] += jnp.sum((x_ref[...] == 0.0).astype(jnp.int32))


def _main_body(keys_ref, a0_ref, a1_ref, x_ref, o_ref):
    i = pl.program_id(0)
    x = x_ref[...]
    rows, cols = x.shape
    row = jax.lax.broadcasted_iota(jnp.int32, (rows, cols), 0)
    col = jax.lax.broadcasted_iota(jnp.int32, (rows, cols), 1)
    idx = ((i * rows + row) * cols + col).astype(jnp.uint32)

    un = _bits_to_unif(_threefry_bits(keys_ref[0], keys_ref[1], idx))
    ub = _bits_to_unif(_threefry_bits(keys_ref[2], keys_ref[3], idx))

    is_zero = x == 0.0
    a = jnp.where(is_zero, a0_ref[...], a1_ref[...])
    p = jax.nn.sigmoid(a - un)
    o_ref[...] = jnp.where(ub < p, 1.0 - x, x)


_STEPS = 5
_ROWS_PER_BLOCK = 8


def kernel(x_start, t):
    B, N = x_start.shape
    grid = (B // _ROWS_PER_BLOCK,)

    count = pl.pallas_call(
        _count_body,
        grid=grid,
        in_specs=[
            pl.BlockSpec((_ROWS_PER_BLOCK, N), lambda i: (i, 0)),
        ],
        out_specs=pl.BlockSpec(
            (1, 1), lambda i: (0, 0), memory_space=pltpu.SMEM
        ),
        out_shape=jax.ShapeDtypeStruct((1, 1), jnp.int32),
    )(x_start)

    sparsity = count[0, 0].astype(jnp.float32) / jnp.float32(B * N)
    gamma_start = 0.1 * (1.0 - sparsity) + 0.001
    gamma_end = gamma_start * 0.1
    epsilon_start = 0.005 * sparsity + 0.0001
    epsilon_end = epsilon_start * 0.1
    gamma = jnp.linspace(gamma_start, gamma_end, _STEPS)
    epsilon = jnp.linspace(epsilon_start, epsilon_end, _STEPS)
    epsilon = jnp.minimum(epsilon, 0.01)
    gamma_cum = 1.0 - jnp.cumprod(1.0 - gamma)
    epsilon_cum = 1.0 - jnp.cumprod(1.0 - epsilon)

    a0 = jnp.take(gamma_cum, t, axis=0)[:, None]
    a1 = jnp.take(epsilon_cum, t, axis=0)[:, None]

    nkd = jax.random.key_data(jax.random.fold_in(jax.random.key(0), 123))
    bkd = jax.random.key_data(jax.random.fold_in(jax.random.key(0), 456))
    keys = jnp.concatenate([nkd, bkd]).astype(jnp.uint32)

    out = pl.pallas_call(
        _main_body,
        grid=grid,
        in_specs=[
            pl.BlockSpec(memory_space=pltpu.SMEM),
            pl.BlockSpec((_ROWS_PER_BLOCK, 1), lambda i: (i, 0)),
            pl.BlockSpec((_ROWS_PER_BLOCK, 1), lambda i: (i, 0)),
            pl.BlockSpec((_ROWS_PER_BLOCK, N), lambda i: (i, 0)),
        ],
        out_specs=pl.BlockSpec((_ROWS_PER_BLOCK, N), lambda i: (i, 0)),
        out_shape=jax.ShapeDtypeStruct((B, N), jnp.float32),
    )(keys, a0, a1, x_start)
    return out


# speedup vs baseline: 1.4208x; 1.4208x over previous
import functools

import jax
import jax.numpy as jnp
from jax.experimental import pallas as pl
from jax.experimental.pallas import tpu as pltpu

_ROT1 = (13, 15, 26, 6)
_ROT2 = (17, 29, 16, 24)


def _four_rounds(x0, x1, rots):
    for r in rots:
        x0 = x0 + x1
        x1 = (x1 << jnp.uint32(r)) | (x1 >> jnp.uint32(32 - r))
        x1 = x1 ^ x0
    return x0, x1


def _threefry_bits(k0, k1, ctr):
    ks2 = k0 ^ k1 ^ jnp.uint32(0x1BD11BDA)
    x0 = k0
    x1 = ctr + k1
    x0, x1 = _four_rounds(x0, x1, _ROT1)
    x0 = x0 + k1
    x1 = x1 + (ks2 + jnp.uint32(1))
    x0, x1 = _four_rounds(x0, x1, _ROT2)
    x0 = x0 + ks2
    x1 = x1 + (k0 + jnp.uint32(2))
    x0, x1 = _four_rounds(x0, x1, _ROT1)
    x0 = x0 + k0
    x1 = x1 + (k1 + jnp.uint32(3))
    x0, x1 = _four_rounds(x0, x1, _ROT2)
    x0 = x0 + k1
    x1 = x1 + (ks2 + jnp.uint32(4))
    x0, x1 = _four_rounds(x0, x1, _ROT1)
    x0 = x0 + ks2
    x1 = x1 + (k0 + jnp.uint32(5))
    return x0 ^ x1


def _bits_to_unif(bits):
    f = jax.lax.bitcast_convert_type(
        (bits >> jnp.uint32(9)) | jnp.uint32(0x3F800000), jnp.float32
    )
    return f - 1.0


def _count_body(x_ref, o_ref):
    @pl.when(pl.program_id(0) == 0)
    def _():
        o_ref[0, 0] = jnp.int32(0)

    o_ref[0, 0] += jnp.sum((x_ref[...] == 0.0).astype(jnp.int32))


def _main_body(n_items, cb, keys_ref, a0_ref, a1_ref, x_ref, o_ref):
    i = pl.program_id(0)
    j = pl.program_id(1)
    x = x_ref[...]
    rows, cols = x.shape
    row_off = (
        jax.lax.broadcasted_iota(jnp.int32, (rows, 1), 0) * n_items
        + (i * rows * n_items + j * cb)
    )
    col = jax.lax.broadcasted_iota(jnp.int32, (rows, cols), 1)
    idx = (row_off + col).astype(jnp.uint32)

    un = _bits_to_unif(_threefry_bits(keys_ref[0], keys_ref[1], idx))
    ub = _bits_to_unif(_threefry_bits(keys_ref[2], keys_ref[3], idx))

    is_zero = x == 0.0
    a = jnp.where(is_zero, a0_ref[...], a1_ref[...])
    p = jax.nn.sigmoid(a - un)
    o_ref[...] = jnp.where(ub < p, 1.0 - x, x)


_STEPS = 5
_ROWS_PER_BLOCK = 8
_COLS_PER_BLOCK = 2048
_COUNT_ROWS = 32


def kernel(x_start, t):
    B, N = x_start.shape

    count = pl.pallas_call(
        _count_body,
        grid=(B // _COUNT_ROWS,),
        in_specs=[
            pl.BlockSpec((_COUNT_ROWS, N), lambda i: (i, 0)),
        ],
        out_specs=pl.BlockSpec(
            (1, 1), lambda i: (0, 0), memory_space=pltpu.SMEM
        ),
        out_shape=jax.ShapeDtypeStruct((1, 1), jnp.int32),
        compiler_params=pltpu.CompilerParams(
            dimension_semantics=("arbitrary",)
        ),
    )(x_start)

    sparsity = count[0, 0].astype(jnp.float32) / jnp.float32(B * N)
    gamma_start = 0.1 * (1.0 - sparsity) + 0.001
    gamma_end = gamma_start * 0.1
    epsilon_start = 0.005 * sparsity + 0.0001
    epsilon_end = epsilon_start * 0.1
    gamma = jnp.linspace(gamma_start, gamma_end, _STEPS)
    epsilon = jnp.linspace(epsilon_start, epsilon_end, _STEPS)
    epsilon = jnp.minimum(epsilon, 0.01)
    gamma_cum = 1.0 - jnp.cumprod(1.0 - gamma)
    epsilon_cum = 1.0 - jnp.cumprod(1.0 - epsilon)

    a0 = jnp.take(gamma_cum, t, axis=0)[:, None]
    a1 = jnp.take(epsilon_cum, t, axis=0)[:, None]

    nkd = jax.random.key_data(jax.random.fold_in(jax.random.key(0), 123))
    bkd = jax.random.key_data(jax.random.fold_in(jax.random.key(0), 456))
    keys = jnp.concatenate([nkd, bkd]).astype(jnp.uint32)

    body = functools.partial(_main_body, N, _COLS_PER_BLOCK)
    out = pl.pallas_call(
        body,
        grid=(B // _ROWS_PER_BLOCK, pl.cdiv(N, _COLS_PER_BLOCK)),
        in_specs=[
            pl.BlockSpec(memory_space=pltpu.SMEM),
            pl.BlockSpec((_ROWS_PER_BLOCK, 1), lambda i, j: (i, 0)),
            pl.BlockSpec((_ROWS_PER_BLOCK, 1), lambda i, j: (i, 0)),
            pl.BlockSpec((_ROWS_PER_BLOCK, _COLS_PER_BLOCK), lambda i, j: (i, j)),
        ],
        out_specs=pl.BlockSpec(
            (_ROWS_PER_BLOCK, _COLS_PER_BLOCK), lambda i, j: (i, j)
        ),
        out_shape=jax.ShapeDtypeStruct((B, N), jnp.float32),
        compiler_params=pltpu.CompilerParams(
            dimension_semantics=("parallel", "parallel")
        ),
    )(keys, a0, a1, x_start)
    return out
